# baseline (speedup 1.0000x reference)
import functools

import jax
import jax.numpy as jnp
from jax import lax
from jax.experimental import pallas as pl
from jax.experimental.pallas import tpu as pltpu
from jax.experimental.pallas import tpu_sc as plsc

HIDDEN = 1024
INTER = 2048
NE = 8
T = 4096

NC = 2
NS = 16
NW = NC * NS
TPW = T // NW
CHUNK = 64
NCHUNK = TPW // CHUNK

SUB = 128
TP = T + NE * SUB
ICCH = 512
NIC = INTER // ICCH


def _router_body(x_ref, rw_ref, pos_ref, w16_ref, offs_ref):
    x = x_ref[...]
    rw = rw_ref[...]
    logits = lax.dot_general(x, rw, (((1,), (1,)), ((), ())),
                             preferred_element_type=jnp.float32)

    v1 = jnp.max(logits, axis=1, keepdims=True)
    eq = (logits == v1).astype(jnp.float32)
    r8 = lax.broadcasted_iota(jnp.int32, (NE, NE), 0)
    c8 = lax.broadcasted_iota(jnp.int32, (NE, NE), 1)
    tri_incl = (r8 <= c8).astype(jnp.float32)
    cum = lax.dot_general(eq, tri_incl, (((1,), (0,)), ((), ())),
                          preferred_element_type=jnp.float32)
    one_hot = eq * (cum == 1.0).astype(jnp.float32)

    neg_inf = jnp.float32(-jnp.inf)
    v2 = jnp.max(jnp.where(one_hot > 0.0, neg_inf, logits), axis=1,
                 keepdims=True)
    w = 1.0 / (1.0 + jnp.exp(v2 - v1))

    counts = jnp.sum(one_hot, axis=0, keepdims=True)
    padded = jnp.ceil(counts * (1.0 / SUB)) * float(SUB)
    r16 = lax.broadcasted_iota(jnp.int32, (NE, 16), 0)
    c16 = lax.broadcasted_iota(jnp.int32, (NE, 16), 1)
    tri16 = (r16 < c16).astype(jnp.float32)
    padded_col = padded.reshape(NE, 1)
    offs16 = jnp.sum(padded_col * tri16, axis=0, keepdims=True)

    CH = 256
    rr = lax.broadcasted_iota(jnp.int32, (CH, CH), 0)
    cc = lax.broadcasted_iota(jnp.int32, (CH, CH), 1)
    ltri = (cc < rr).astype(jnp.float32)
    ranks = []
    carry = jnp.zeros((1, NE), jnp.float32)
    for k in range(T // CH):
        oh = lax.slice(one_hot, (k * CH, 0), ((k + 1) * CH, NE))
        rk = lax.dot_general(ltri, oh, (((1,), (0,)), ((), ())),
                             preferred_element_type=jnp.float32) + carry
        ranks.append(rk)
        carry = carry + jnp.sum(oh, axis=0, keepdims=True)
    rank = jnp.concatenate(ranks, axis=0)

    offs8 = lax.slice(offs16, (0, 0), (1, NE))
    pos_f = jnp.sum(one_hot * (rank + offs8), axis=1, keepdims=True)
    pos_ref[...] = pos_f.astype(jnp.int32)
    w16_ref[...] = jnp.broadcast_to(w, (T, 128))
    offs_ref[...] = offs16.astype(jnp.int32)


def _router_call(xf, router_w):
    return pl.pallas_call(
        _router_body,
        out_shape=[
            jax.ShapeDtypeStruct((T, 1), jnp.int32),
            jax.ShapeDtypeStruct((T, 128), jnp.float32),
            jax.ShapeDtypeStruct((1, 16), jnp.int32),
        ],
    )(xf, router_w)


def _dispatch_body(x_hbm, w16_hbm, pos_hbm, xs_hbm, ws_hbm,
                   idx_v, xrows_v, wrows_v):
    wid = lax.axis_index("s") * NC + lax.axis_index("c")
    base0 = wid * TPW
    for c in range(NCHUNK):
        base = base0 + c * CHUNK
        pltpu.sync_copy(pos_hbm.at[pl.ds(base, CHUNK)], idx_v)
        pltpu.sync_copy(x_hbm.at[pl.ds(base, CHUNK)], xrows_v)
        pltpu.sync_copy(xrows_v, xs_hbm.at[idx_v])
        pltpu.sync_copy(w16_hbm.at[pl.ds(base, CHUNK)], wrows_v)
        pltpu.sync_copy(wrows_v, ws_hbm.at[idx_v])


def _dispatch_call(xf, w16, pos1):
    mesh = plsc.VectorSubcoreMesh(core_axis_name="c", subcore_axis_name="s")
    f = pl.kernel(
        _dispatch_body,
        out_type=[
            jax.ShapeDtypeStruct((TP, HIDDEN), jnp.float32),
            jax.ShapeDtypeStruct((TP, 128), jnp.float32),
        ],
        mesh=mesh,
        scratch_types=[
            pltpu.VMEM((CHUNK,), jnp.int32),
            pltpu.VMEM((CHUNK, HIDDEN), jnp.float32),
            pltpu.VMEM((CHUNK, 128), jnp.float32),
        ],
    )
    return f(xf, w16, pos1)


def _ffn_body(offs_ref, xs_ref, ws_ref, gate_ref, up_ref, down_ref, out_ref):
    e = pl.program_id(0)
    ic = pl.program_id(1)
    start_e = offs_ref[e]
    end_e = offs_ref[e + 1]
    n_sub = lax.div(end_e - start_e, SUB)

    gate_b = gate_ref[0]
    up_b = up_ref[0]
    down_b = down_ref[0]
    is_first = ic == 0
    is_last = ic == NIC - 1

    def body(j, _):
        start = pl.multiple_of(start_e + j * SUB, SUB)
        x_sub = xs_ref[pl.ds(start, SUB), :]
        g = lax.dot_general(x_sub, gate_b, (((1,), (1,)), ((), ())),
                            preferred_element_type=jnp.float32)
        u = lax.dot_general(x_sub, up_b, (((1,), (1,)), ((), ())),
                            preferred_element_type=jnp.float32)
        a = g * (1.0 / (1.0 + jnp.exp(-g))) * u
        y = lax.dot_general(a, down_b, (((1,), (1,)), ((), ())),
                            preferred_element_type=jnp.float32)

        @pl.when(is_first)
        def _():
            out_ref[pl.ds(start, SUB), :] = y

        @pl.when(jnp.logical_and(jnp.logical_not(is_first),
                                 jnp.logical_not(is_last)))
        def _():
            out_ref[pl.ds(start, SUB), :] += y

        @pl.when(jnp.logical_and(jnp.logical_not(is_first), is_last))
        def _():
            wrow = ws_ref[pl.ds(start, SUB), :][:, 0:1]
            out_ref[pl.ds(start, SUB), :] = (
                out_ref[pl.ds(start, SUB), :] + y) * wrow

        return 0

    lax.fori_loop(0, n_sub, body, 0)


def _ffn_call(offs16, xs, ws, gate_w, up_w, down_w):
    return pl.pallas_call(
        _ffn_body,
        grid=(NE, NIC),
        in_specs=[
            pl.BlockSpec(memory_space=pltpu.SMEM),
            pl.BlockSpec((TP, HIDDEN), lambda e, ic: (0, 0)),
            pl.BlockSpec((TP, 128), lambda e, ic: (0, 0)),
            pl.BlockSpec((1, ICCH, HIDDEN), lambda e, ic: (e, ic, 0)),
            pl.BlockSpec((1, ICCH, HIDDEN), lambda e, ic: (e, ic, 0)),
            pl.BlockSpec((1, HIDDEN, ICCH), lambda e, ic: (e, 0, ic)),
        ],
        out_specs=pl.BlockSpec((TP, HIDDEN), lambda e, ic: (0, 0)),
        out_shape=jax.ShapeDtypeStruct((TP, HIDDEN), jnp.float32),
        compiler_params=pltpu.CompilerParams(
            dimension_semantics=("arbitrary", "arbitrary"),
            vmem_limit_bytes=62 * 1024 * 1024,
        ),
    )(offs16, xs, ws, gate_w, up_w, down_w)


def _combine_body(ys_hbm, pos_hbm, out_hbm, idx_v, rows_v):
    wid = lax.axis_index("s") * NC + lax.axis_index("c")
    base0 = wid * TPW
    for c in range(NCHUNK):
        base = base0 + c * CHUNK
        pltpu.sync_copy(pos_hbm.at[pl.ds(base, CHUNK)], idx_v)
        pltpu.sync_copy(ys_hbm.at[idx_v], rows_v)
        pltpu.sync_copy(rows_v, out_hbm.at[pl.ds(base, CHUNK)])


def _combine_call(ys, pos1):
    mesh = plsc.VectorSubcoreMesh(core_axis_name="c", subcore_axis_name="s")
    f = pl.kernel(
        _combine_body,
        out_type=jax.ShapeDtypeStruct((T, HIDDEN), jnp.float32),
        mesh=mesh,
        scratch_types=[
            pltpu.VMEM((CHUNK,), jnp.int32),
            pltpu.VMEM((CHUNK, HIDDEN), jnp.float32),
        ],
    )
    return f(ys, pos1)


def kernel(hidden_states, router_w, gate_w, up_w, down_w):
    bsz, seq, h = hidden_states.shape
    xf = hidden_states.reshape(T, h)
    pos, w16, offs = _router_call(xf, router_w)
    pos1 = pos.reshape(T)
    offs16 = offs.reshape(16)
    xs, ws = _dispatch_call(xf, w16, pos1)
    ys = _ffn_call(offs16, xs, ws, gate_w, up_w, down_w)
    out = _combine_call(ys, pos1)
    return out.reshape(bsz, seq, h)


# speedup vs baseline: 1.4220x; 1.4220x over previous
import functools

import jax
import jax.numpy as jnp
from jax import lax
from jax.experimental import pallas as pl
from jax.experimental.pallas import tpu as pltpu
from jax.experimental.pallas import tpu_sc as plsc

HIDDEN = 1024
INTER = 2048
NE = 8
T = 4096

NC = 2
NS = 16
NW = NC * NS
TPW = T // NW
CHUNK = 64
NCHUNK = TPW // CHUNK

SUB = 256
TP = T + NE * SUB
NT = TP // SUB


def _router_body(x_ref, rw_ref, pos_ref, w16_ref, eid_ref):
    x = x_ref[...]
    rw = rw_ref[...]
    logits = lax.dot_general(x, rw, (((1,), (1,)), ((), ())),
                             preferred_element_type=jnp.float32)

    v1 = jnp.max(logits, axis=1, keepdims=True)
    eq = (logits == v1).astype(jnp.float32)
    r8 = lax.broadcasted_iota(jnp.int32, (NE, NE), 0)
    c8 = lax.broadcasted_iota(jnp.int32, (NE, NE), 1)
    tri_incl = (r8 <= c8).astype(jnp.float32)
    cum = lax.dot_general(eq, tri_incl, (((1,), (0,)), ((), ())),
                          preferred_element_type=jnp.float32)
    one_hot = eq * (cum == 1.0).astype(jnp.float32)

    neg_inf = jnp.float32(-jnp.inf)
    v2 = jnp.max(jnp.where(one_hot > 0.0, neg_inf, logits), axis=1,
                 keepdims=True)
    w = 1.0 / (1.0 + jnp.exp(v2 - v1))

    counts = jnp.sum(one_hot, axis=0, keepdims=True)
    padded = jnp.ceil(counts * (1.0 / SUB)) * float(SUB)
    r16 = lax.broadcasted_iota(jnp.int32, (NE, 16), 0)
    c16 = lax.broadcasted_iota(jnp.int32, (NE, 16), 1)
    tri16 = (r16 < c16).astype(jnp.float32)
    padded_col = padded.reshape(NE, 1)
    offs16 = jnp.sum(padded_col * tri16, axis=0, keepdims=True)

    CH = 256
    rr = lax.broadcasted_iota(jnp.int32, (CH, CH), 0)
    cc = lax.broadcasted_iota(jnp.int32, (CH, CH), 1)
    ltri = (cc < rr).astype(jnp.float32)
    ranks = []
    carry = jnp.zeros((1, NE), jnp.float32)
    for k in range(T // CH):
        oh = lax.slice(one_hot, (k * CH, 0), ((k + 1) * CH, NE))
        rk = lax.dot_general(ltri, oh, (((1,), (0,)), ((), ())),
                             preferred_element_type=jnp.float32) + carry
        ranks.append(rk)
        carry = carry + jnp.sum(oh, axis=0, keepdims=True)
    rank = jnp.concatenate(ranks, axis=0)

    offs8 = lax.slice(offs16, (0, 0), (1, NE))
    pos_f = jnp.sum(one_hot * (rank + offs8), axis=1, keepdims=True)
    pos_ref[...] = pos_f.astype(jnp.int32)
    w16_ref[...] = jnp.broadcast_to(w, (T, 128))

    ge8 = (r8 >= c8).astype(jnp.float32)
    padded_row8 = jnp.broadcast_to(padded, (NE, NE))
    ends_col = jnp.sum(padded_row8 * ge8, axis=1, keepdims=True)
    ts = (lax.broadcasted_iota(jnp.int32, (1, 32), 1)
          * SUB).astype(jnp.float32)
    eid_f = jnp.sum((ends_col <= ts).astype(jnp.float32), axis=0,
                    keepdims=True)
    eid_ref[...] = jnp.minimum(eid_f, float(NE - 1)).astype(jnp.int32)


def _router_call(xf, router_w):
    return pl.pallas_call(
        _router_body,
        out_shape=[
            jax.ShapeDtypeStruct((T, 1), jnp.int32),
            jax.ShapeDtypeStruct((T, 128), jnp.float32),
            jax.ShapeDtypeStruct((1, 32), jnp.int32),
        ],
    )(xf, router_w)


def _dispatch_body(x_hbm, w16_hbm, pos_hbm, xs_hbm, ws_hbm,
                   idx_v, xrows_v, wrows_v):
    wid = lax.axis_index("s") * NC + lax.axis_index("c")
    base0 = wid * TPW
    for c in range(NCHUNK):
        base = base0 + c * CHUNK
        pltpu.sync_copy(pos_hbm.at[pl.ds(base, CHUNK)], idx_v)
        pltpu.sync_copy(x_hbm.at[pl.ds(base, CHUNK)], xrows_v)
        pltpu.sync_copy(xrows_v, xs_hbm.at[idx_v])
        pltpu.sync_copy(w16_hbm.at[pl.ds(base, CHUNK)], wrows_v)
        pltpu.sync_copy(wrows_v, ws_hbm.at[idx_v])


def _dispatch_call(xf, w16, pos1):
    mesh = plsc.VectorSubcoreMesh(core_axis_name="c", subcore_axis_name="s")
    f = pl.kernel(
        _dispatch_body,
        out_type=[
            jax.ShapeDtypeStruct((TP, HIDDEN), jnp.float32),
            jax.ShapeDtypeStruct((TP, 128), jnp.float32),
        ],
        mesh=mesh,
        scratch_types=[
            pltpu.VMEM((CHUNK,), jnp.int32),
            pltpu.VMEM((CHUNK, HIDDEN), jnp.float32),
            pltpu.VMEM((CHUNK, 128), jnp.float32),
        ],
    )
    return f(xf, w16, pos1)


def _ffn_body(eid_ref, x_ref, ws_ref, gate_ref, up_ref, down_ref, out_ref):
    x = x_ref[...]
    g = lax.dot_general(x, gate_ref[0], (((1,), (1,)), ((), ())),
                        preferred_element_type=jnp.float32)
    u = lax.dot_general(x, up_ref[0], (((1,), (1,)), ((), ())),
                        preferred_element_type=jnp.float32)
    a = g * (1.0 / (1.0 + jnp.exp(-g))) * u
    y = lax.dot_general(a, down_ref[0], (((1,), (1,)), ((), ())),
                        preferred_element_type=jnp.float32)
    out_ref[...] = y * ws_ref[:, 0:1]


def _ffn_call(eid32, xs, ws, gate_w, up_w, down_w):
    grid_spec = pltpu.PrefetchScalarGridSpec(
        num_scalar_prefetch=1,
        grid=(NT,),
        in_specs=[
            pl.BlockSpec((SUB, HIDDEN), lambda m, eid: (m, 0)),
            pl.BlockSpec((SUB, 128), lambda m, eid: (m, 0)),
            pl.BlockSpec((1, INTER, HIDDEN), lambda m, eid: (eid[m], 0, 0)),
            pl.BlockSpec((1, INTER, HIDDEN), lambda m, eid: (eid[m], 0, 0)),
            pl.BlockSpec((1, HIDDEN, INTER), lambda m, eid: (eid[m], 0, 0)),
        ],
        out_specs=pl.BlockSpec((SUB, HIDDEN), lambda m, eid: (m, 0)),
    )
    return pl.pallas_call(
        _ffn_body,
        grid_spec=grid_spec,
        out_shape=jax.ShapeDtypeStruct((TP, HIDDEN), jnp.float32),
        compiler_params=pltpu.CompilerParams(
            dimension_semantics=("arbitrary",),
            vmem_limit_bytes=63 * 1024 * 1024,
        ),
    )(eid32, xs, ws, gate_w, up_w, down_w)


def _combine_body(ys_hbm, pos_hbm, out_hbm, idx_v, rows_v):
    wid = lax.axis_index("s") * NC + lax.axis_index("c")
    base0 = wid * TPW
    for c in range(NCHUNK):
        base = base0 + c * CHUNK
        pltpu.sync_copy(pos_hbm.at[pl.ds(base, CHUNK)], idx_v)
        pltpu.sync_copy(ys_hbm.at[idx_v], rows_v)
        pltpu.sync_copy(rows_v, out_hbm.at[pl.ds(base, CHUNK)])


def _combine_call(ys, pos1):
    mesh = plsc.VectorSubcoreMesh(core_axis_name="c", subcore_axis_name="s")
    f = pl.kernel(
        _combine_body,
        out_type=jax.ShapeDtypeStruct((T, HIDDEN), jnp.float32),
        mesh=mesh,
        scratch_types=[
            pltpu.VMEM((CHUNK,), jnp.int32),
            pltpu.VMEM((CHUNK, HIDDEN), jnp.float32),
        ],
    )
    return f(ys, pos1)


def kernel(hidden_states, router_w, gate_w, up_w, down_w):
    bsz, seq, h = hidden_states.shape
    xf = hidden_states.reshape(T, h)
    pos, w16, eid = _router_call(xf, router_w)
    pos1 = pos.reshape(T)
    eid32 = eid.reshape(32)
    xs, ws = _dispatch_call(xf, w16, pos1)
    ys = _ffn_call(eid32, xs, ws, gate_w, up_w, down_w)
    out = _combine_call(ys, pos1)
    return out.reshape(bsz, seq, h)


# speedup vs baseline: 1.7207x; 1.2100x over previous
import functools

import jax
import jax.numpy as jnp
from jax import lax
from jax.experimental import pallas as pl
from jax.experimental.pallas import tpu as pltpu
from jax.experimental.pallas import tpu_sc as plsc

HIDDEN = 1024
INTER = 2048
NE = 8
T = 4096

NC = 2
NS = 16
NW = NC * NS
TPW = T // NW
CHUNK = 64
NCHUNK = TPW // CHUNK

SUB = 256
TP = T + NE * SUB
NT = TP // SUB


def _router_body(x_ref, rw_ref, pos_ref, w16_ref, meta_ref):
    x = x_ref[...]
    rw = rw_ref[...]
    logits = lax.dot_general(x, rw, (((1,), (1,)), ((), ())),
                             preferred_element_type=jnp.float32)

    v1 = jnp.max(logits, axis=1, keepdims=True)
    eq = (logits == v1).astype(jnp.float32)
    r8 = lax.broadcasted_iota(jnp.int32, (NE, NE), 0)
    c8 = lax.broadcasted_iota(jnp.int32, (NE, NE), 1)
    tri_incl = (r8 <= c8).astype(jnp.float32)
    cum = lax.dot_general(eq, tri_incl, (((1,), (0,)), ((), ())),
                          preferred_element_type=jnp.float32)
    one_hot = eq * (cum == 1.0).astype(jnp.float32)

    neg_inf = jnp.float32(-jnp.inf)
    v2 = jnp.max(jnp.where(one_hot > 0.0, neg_inf, logits), axis=1,
                 keepdims=True)
    w = 1.0 / (1.0 + jnp.exp(v2 - v1))

    counts = jnp.sum(one_hot, axis=0, keepdims=True)
    padded = jnp.ceil(counts * (1.0 / SUB)) * float(SUB)
    r16 = lax.broadcasted_iota(jnp.int32, (NE, 16), 0)
    c16 = lax.broadcasted_iota(jnp.int32, (NE, 16), 1)
    tri16 = (r16 < c16).astype(jnp.float32)
    padded_col = padded.reshape(NE, 1)
    offs16 = jnp.sum(padded_col * tri16, axis=0, keepdims=True)

    CH = 256
    rr = lax.broadcasted_iota(jnp.int32, (CH, CH), 0)
    cc = lax.broadcasted_iota(jnp.int32, (CH, CH), 1)
    ltri = (cc < rr).astype(jnp.float32)
    ranks = []
    carry = jnp.zeros((1, NE), jnp.float32)
    for k in range(T // CH):
        oh = lax.slice(one_hot, (k * CH, 0), ((k + 1) * CH, NE))
        rk = lax.dot_general(ltri, oh, (((1,), (0,)), ((), ())),
                             preferred_element_type=jnp.float32) + carry
        ranks.append(rk)
        carry = carry + jnp.sum(oh, axis=0, keepdims=True)
    rank = jnp.concatenate(ranks, axis=0)

    offs8 = lax.slice(offs16, (0, 0), (1, NE))
    pos_f = jnp.sum(one_hot * (rank + offs8), axis=1, keepdims=True)
    pos_ref[...] = pos_f.astype(jnp.int32)
    w16_ref[...] = jnp.broadcast_to(w, (T, 128))

    ge8 = (r8 >= c8).astype(jnp.float32)
    padded_row8 = jnp.broadcast_to(padded, (NE, NE))
    ends_col = jnp.sum(padded_row8 * ge8, axis=1, keepdims=True)
    ts = (lax.broadcasted_iota(jnp.int32, (1, 32), 1)
          * SUB).astype(jnp.float32)
    eid_raw = jnp.sum((ends_col <= ts).astype(jnp.float32), axis=0,
                      keepdims=True)
    total_pad = jnp.sum(padded, axis=1, keepdims=True)
    skip = (ts >= total_pad).astype(jnp.float32)
    last_eid = jnp.sum((ends_col <= total_pad - SUB).astype(jnp.float32),
                       axis=0, keepdims=True)
    eid = jnp.minimum(eid_raw, last_eid)

    def shift_right_lanes(a, n, fill):
        pad = jnp.full((1, n), fill, jnp.float32)
        return jnp.concatenate([pad, lax.slice(a, (0, 0), (1, 32 - n))],
                               axis=1)

    def shift_left_lanes(a, n, fill):
        pad = jnp.full((1, n), fill, jnp.float32)
        return jnp.concatenate([lax.slice(a, (0, n), (1, 32)), pad],
                               axis=1)

    prev_eid = shift_right_lanes(eid, 1, -1.0)
    newflag = (eid != prev_eid).astype(jnp.float32)
    cum = newflag
    for sh in (1, 2, 4, 8, 16):
        cum = cum + shift_right_lanes(cum, sh, 0.0)
    rankd = cum - 1.0
    dpar = rankd - 2.0 * jnp.floor(rankd * 0.5)
    BIG = 99.0
    chg = jnp.where(newflag > 0.0, eid, BIG)
    nxt = shift_left_lanes(chg, 1, BIG)
    for sh in (1, 2, 4, 8, 16):
        nxt = jnp.minimum(nxt, shift_left_lanes(nxt, sh, BIG))
    meta = jnp.concatenate([eid, newflag, dpar, nxt, skip], axis=0)
    meta_ref[...] = meta.astype(jnp.int32)


def _router_call(xf, router_w):
    return pl.pallas_call(
        _router_body,
        out_shape=[
            jax.ShapeDtypeStruct((T, 1), jnp.int32),
            jax.ShapeDtypeStruct((T, 128), jnp.float32),
            jax.ShapeDtypeStruct((5, 32), jnp.int32),
        ],
    )(xf, router_w)


def _dispatch_body(x_hbm, w16_hbm, pos_hbm, xs_hbm, ws_hbm,
                   idx_v, xrows_v, wrows_v):
    wid = lax.axis_index("s") * NC + lax.axis_index("c")
    base0 = wid * TPW
    for c in range(NCHUNK):
        base = base0 + c * CHUNK
        pltpu.sync_copy(pos_hbm.at[pl.ds(base, CHUNK)], idx_v)
        pltpu.sync_copy(x_hbm.at[pl.ds(base, CHUNK)], xrows_v)
        pltpu.sync_copy(xrows_v, xs_hbm.at[idx_v])
        pltpu.sync_copy(w16_hbm.at[pl.ds(base, CHUNK)], wrows_v)
        pltpu.sync_copy(wrows_v, ws_hbm.at[idx_v])


def _dispatch_call(xf, w16, pos1):
    mesh = plsc.VectorSubcoreMesh(core_axis_name="c", subcore_axis_name="s")
    f = pl.kernel(
        _dispatch_body,
        out_type=[
            jax.ShapeDtypeStruct((TP, HIDDEN), jnp.float32),
            jax.ShapeDtypeStruct((TP, 128), jnp.float32),
        ],
        mesh=mesh,
        scratch_types=[
            pltpu.VMEM((CHUNK,), jnp.int32),
            pltpu.VMEM((CHUNK, HIDDEN), jnp.float32),
            pltpu.VMEM((CHUNK, 128), jnp.float32),
        ],
    )
    return f(xf, w16, pos1)


def _ffn_body(meta_ref, x_ref, ws_ref, gate_hbm, up_hbm, down_hbm, out_ref,
              g0, u0, d0, g1, u1, d1, sem0, sem1):
    m = pl.program_id(0)
    eid = meta_ref[0, m]
    newf = meta_ref[1, m]
    dpar = meta_ref[2, m]
    nxt = meta_ref[3, m]
    skip = meta_ref[4, m]

    def issue(e, gb, ub, db, sem):
        pltpu.make_async_copy(gate_hbm.at[e], gb, sem).start()
        pltpu.make_async_copy(up_hbm.at[e], ub, sem).start()
        pltpu.make_async_copy(down_hbm.at[e], db, sem).start()

    def wait(e, gb, ub, db, sem):
        pltpu.make_async_copy(gate_hbm.at[e], gb, sem).wait()
        pltpu.make_async_copy(up_hbm.at[e], ub, sem).wait()
        pltpu.make_async_copy(down_hbm.at[e], db, sem).wait()

    @pl.when(m == 0)
    def _():
        issue(eid, g0, u0, d0, sem0)

    @pl.when(jnp.logical_and(newf == 1, dpar == 0))
    def _():
        wait(eid, g0, u0, d0, sem0)

        @pl.when(nxt < NE)
        def _():
            issue(nxt, g1, u1, d1, sem1)

    @pl.when(jnp.logical_and(newf == 1, dpar == 1))
    def _():
        wait(eid, g1, u1, d1, sem1)

        @pl.when(nxt < NE)
        def _():
            issue(nxt, g0, u0, d0, sem0)

    def tile(gb, ub, db):
        x = x_ref[...]
        g = lax.dot_general(x, gb[...], (((1,), (1,)), ((), ())),
                            preferred_element_type=jnp.float32)
        u = lax.dot_general(x, ub[...], (((1,), (1,)), ((), ())),
                            preferred_element_type=jnp.float32)
        a = g * (1.0 / (1.0 + jnp.exp(-g))) * u
        y = lax.dot_general(a, db[...], (((1,), (1,)), ((), ())),
                            preferred_element_type=jnp.float32)
        out_ref[...] = y * ws_ref[:, 0:1]

    @pl.when(jnp.logical_and(skip == 0, dpar == 0))
    def _():
        tile(g0, u0, d0)

    @pl.when(jnp.logical_and(skip == 0, dpar == 1))
    def _():
        tile(g1, u1, d1)


def _ffn_call(meta, xs, ws, gate_w, up_w, down_w):
    grid_spec = pltpu.PrefetchScalarGridSpec(
        num_scalar_prefetch=1,
        grid=(NT,),
        in_specs=[
            pl.BlockSpec((SUB, HIDDEN), lambda m, meta: (m, 0)),
            pl.BlockSpec((SUB, 128), lambda m, meta: (m, 0)),
            pl.BlockSpec(memory_space=pl.ANY),
            pl.BlockSpec(memory_space=pl.ANY),
            pl.BlockSpec(memory_space=pl.ANY),
        ],
        out_specs=pl.BlockSpec((SUB, HIDDEN), lambda m, meta: (m, 0)),
        scratch_shapes=[
            pltpu.VMEM((INTER, HIDDEN), jnp.float32),
            pltpu.VMEM((INTER, HIDDEN), jnp.float32),
            pltpu.VMEM((HIDDEN, INTER), jnp.float32),
            pltpu.VMEM((INTER, HIDDEN), jnp.float32),
            pltpu.VMEM((INTER, HIDDEN), jnp.float32),
            pltpu.VMEM((HIDDEN, INTER), jnp.float32),
            pltpu.SemaphoreType.DMA,
            pltpu.SemaphoreType.DMA,
        ],
    )
    return pl.pallas_call(
        _ffn_body,
        grid_spec=grid_spec,
        out_shape=jax.ShapeDtypeStruct((TP, HIDDEN), jnp.float32),
        compiler_params=pltpu.CompilerParams(
            dimension_semantics=("arbitrary",),
            vmem_limit_bytes=63 * 1024 * 1024,
        ),
    )(meta, xs, ws, gate_w, up_w, down_w)


def _combine_body(ys_hbm, pos_hbm, out_hbm, idx_v, rows_v):
    wid = lax.axis_index("s") * NC + lax.axis_index("c")
    base0 = wid * TPW
    for c in range(NCHUNK):
        base = base0 + c * CHUNK
        pltpu.sync_copy(pos_hbm.at[pl.ds(base, CHUNK)], idx_v)
        pltpu.sync_copy(ys_hbm.at[idx_v], rows_v)
        pltpu.sync_copy(rows_v, out_hbm.at[pl.ds(base, CHUNK)])


def _combine_call(ys, pos1):
    mesh = plsc.VectorSubcoreMesh(core_axis_name="c", subcore_axis_name="s")
    f = pl.kernel(
        _combine_body,
        out_type=jax.ShapeDtypeStruct((T, HIDDEN), jnp.float32),
        mesh=mesh,
        scratch_types=[
            pltpu.VMEM((CHUNK,), jnp.int32),
            pltpu.VMEM((CHUNK, HIDDEN), jnp.float32),
        ],
    )
    return f(ys, pos1)


def kernel(hidden_states, router_w, gate_w, up_w, down_w):
    bsz, seq, h = hidden_states.shape
    xf = hidden_states.reshape(T, h)
    pos, w16, meta = _router_call(xf, router_w)
    pos1 = pos.reshape(T)
    xs, ws = _dispatch_call(xf, w16, pos1)
    ys = _ffn_call(meta, xs, ws, gate_w, up_w, down_w)
    out = _combine_call(ys, pos1)
    return out.reshape(bsz, seq, h)


# speedup vs baseline: 1.7476x; 1.0156x over previous
import functools

import jax
import jax.numpy as jnp
from jax import lax
from jax.experimental import pallas as pl
from jax.experimental.pallas import tpu as pltpu
from jax.experimental.pallas import tpu_sc as plsc

HIDDEN = 1024
INTER = 2048
NE = 8
T = 4096

NC = 2
NS = 16
NW = NC * NS
TPW = T // NW
CHUNK = 64
NCHUNK = TPW // CHUNK

SUB = 256
TP = T + NE * SUB
NT = TP // SUB


def _router_body(x_ref, rw_ref, pos_ref, w16_ref, meta_ref):
    x = x_ref[...]
    rw = rw_ref[...]
    logits = lax.dot_general(x, rw, (((1,), (1,)), ((), ())),
                             preferred_element_type=jnp.float32)

    v1 = jnp.max(logits, axis=1, keepdims=True)
    eq = (logits == v1).astype(jnp.float32)
    r8 = lax.broadcasted_iota(jnp.int32, (NE, NE), 0)
    c8 = lax.broadcasted_iota(jnp.int32, (NE, NE), 1)
    tri_incl = (r8 <= c8).astype(jnp.float32)
    cum = lax.dot_general(eq, tri_incl, (((1,), (0,)), ((), ())),
                          preferred_element_type=jnp.float32)
    one_hot = eq * (cum == 1.0).astype(jnp.float32)

    neg_inf = jnp.float32(-jnp.inf)
    v2 = jnp.max(jnp.where(one_hot > 0.0, neg_inf, logits), axis=1,
                 keepdims=True)
    w = 1.0 / (1.0 + jnp.exp(v2 - v1))

    counts = jnp.sum(one_hot, axis=0, keepdims=True)
    padded = jnp.ceil(counts * (1.0 / SUB)) * float(SUB)
    r16 = lax.broadcasted_iota(jnp.int32, (NE, 16), 0)
    c16 = lax.broadcasted_iota(jnp.int32, (NE, 16), 1)
    tri16 = (r16 < c16).astype(jnp.float32)
    padded_col = padded.reshape(NE, 1)
    offs16 = jnp.sum(padded_col * tri16, axis=0, keepdims=True)

    CH = 256
    rr = lax.broadcasted_iota(jnp.int32, (CH, CH), 0)
    cc = lax.broadcasted_iota(jnp.int32, (CH, CH), 1)
    ltri = (cc < rr).astype(jnp.float32)
    ranks = []
    carry = jnp.zeros((1, NE), jnp.float32)
    for k in range(T // CH):
        oh = lax.slice(one_hot, (k * CH, 0), ((k + 1) * CH, NE))
        rk = lax.dot_general(ltri, oh, (((1,), (0,)), ((), ())),
                             preferred_element_type=jnp.float32) + carry
        ranks.append(rk)
        carry = carry + jnp.sum(oh, axis=0, keepdims=True)
    rank = jnp.concatenate(ranks, axis=0)

    offs8 = lax.slice(offs16, (0, 0), (1, NE))
    pos_f = jnp.sum(one_hot * (rank + offs8), axis=1, keepdims=True)
    pos_ref[...] = pos_f.astype(jnp.int32).reshape(T)
    w16_ref[...] = jnp.broadcast_to(w, (T, 128))

    ge8 = (r8 >= c8).astype(jnp.float32)
    padded_row8 = jnp.broadcast_to(padded, (NE, NE))
    ends_col = jnp.sum(padded_row8 * ge8, axis=1, keepdims=True)
    ts = (lax.broadcasted_iota(jnp.int32, (1, 32), 1)
          * SUB).astype(jnp.float32)
    eid_raw = jnp.sum((ends_col <= ts).astype(jnp.float32), axis=0,
                      keepdims=True)
    total_pad = jnp.sum(padded, axis=1, keepdims=True)
    skip = (ts >= total_pad).astype(jnp.float32)
    last_eid = jnp.sum((ends_col <= total_pad - SUB).astype(jnp.float32),
                       axis=0, keepdims=True)
    eid = jnp.minimum(eid_raw, last_eid)

    def shift_right_lanes(a, n, fill):
        pad = jnp.full((1, n), fill, jnp.float32)
        return jnp.concatenate([pad, lax.slice(a, (0, 0), (1, 32 - n))],
                               axis=1)

    def shift_left_lanes(a, n, fill):
        pad = jnp.full((1, n), fill, jnp.float32)
        return jnp.concatenate([lax.slice(a, (0, n), (1, 32)), pad],
                               axis=1)

    prev_eid = shift_right_lanes(eid, 1, -1.0)
    newflag = (eid != prev_eid).astype(jnp.float32)
    cum = newflag
    for sh in (1, 2, 4, 8, 16):
        cum = cum + shift_right_lanes(cum, sh, 0.0)
    rankd = cum - 1.0
    dpar = rankd - 2.0 * jnp.floor(rankd * 0.5)
    BIG = 99.0
    chg = jnp.where(newflag > 0.0, eid, BIG)
    nxt = shift_left_lanes(chg, 1, BIG)
    for sh in (1, 2, 4, 8, 16):
        nxt = jnp.minimum(nxt, shift_left_lanes(nxt, sh, BIG))
    meta = jnp.concatenate([eid, newflag, dpar, nxt, skip], axis=0)
    meta_ref[...] = meta.astype(jnp.int32)


def _router_call(xf, router_w):
    return pl.pallas_call(
        _router_body,
        out_shape=[
            jax.ShapeDtypeStruct((T,), jnp.int32),
            jax.ShapeDtypeStruct((T, 128), jnp.float32),
            jax.ShapeDtypeStruct((5, 32), jnp.int32),
        ],
    )(xf, router_w)


def _dispatch_body(x_hbm, w16_hbm, pos_hbm, xs_hbm, ws_hbm,
                   idx_v, xrows_v, wrows_v, sem):
    wid = lax.axis_index("s") * NC + lax.axis_index("c")
    base0 = wid * TPW
    for c in range(NCHUNK):
        base = base0 + c * CHUNK
        ld = [
            pltpu.make_async_copy(pos_hbm.at[pl.ds(base, CHUNK)], idx_v, sem),
            pltpu.make_async_copy(x_hbm.at[pl.ds(base, CHUNK)], xrows_v, sem),
            pltpu.make_async_copy(w16_hbm.at[pl.ds(base, CHUNK)], wrows_v,
                                  sem),
        ]
        for cp in ld:
            cp.start()
        for cp in ld:
            cp.wait()
        st = [
            pltpu.make_async_copy(xrows_v, xs_hbm.at[idx_v], sem),
            pltpu.make_async_copy(wrows_v, ws_hbm.at[idx_v], sem),
        ]
        for cp in st:
            cp.start()
        for cp in st:
            cp.wait()


def _dispatch_call(xf, w16, pos1):
    mesh = plsc.VectorSubcoreMesh(core_axis_name="c", subcore_axis_name="s")
    f = pl.kernel(
        _dispatch_body,
        out_type=[
            jax.ShapeDtypeStruct((TP, HIDDEN), jnp.float32),
            jax.ShapeDtypeStruct((TP, 128), jnp.float32),
        ],
        mesh=mesh,
        scratch_types=[
            pltpu.VMEM((CHUNK,), jnp.int32),
            pltpu.VMEM((CHUNK, HIDDEN), jnp.float32),
            pltpu.VMEM((CHUNK, 128), jnp.float32),
            pltpu.SemaphoreType.DMA,
        ],
    )
    return f(xf, w16, pos1)


def _ffn_body(meta_ref, x_ref, ws_ref, gate_hbm, up_hbm, down_hbm, out_ref,
              g0, u0, d0, g1, u1, d1, sem0, sem1):
    m = pl.program_id(0)
    eid = meta_ref[0, m]
    newf = meta_ref[1, m]
    dpar = meta_ref[2, m]
    nxt = meta_ref[3, m]
    skip = meta_ref[4, m]

    HI = INTER // 2
    HH = HIDDEN // 2

    def copies(e, gb, ub, db, sem):
        return [
            pltpu.make_async_copy(gate_hbm.at[e, pl.ds(0, HI)],
                                  gb.at[pl.ds(0, HI)], sem),
            pltpu.make_async_copy(gate_hbm.at[e, pl.ds(HI, HI)],
                                  gb.at[pl.ds(HI, HI)], sem),
            pltpu.make_async_copy(up_hbm.at[e, pl.ds(0, HI)],
                                  ub.at[pl.ds(0, HI)], sem),
            pltpu.make_async_copy(up_hbm.at[e, pl.ds(HI, HI)],
                                  ub.at[pl.ds(HI, HI)], sem),
            pltpu.make_async_copy(down_hbm.at[e, pl.ds(0, HH)],
                                  db.at[pl.ds(0, HH)], sem),
            pltpu.make_async_copy(down_hbm.at[e, pl.ds(HH, HH)],
                                  db.at[pl.ds(HH, HH)], sem),
        ]

    def issue(e, gb, ub, db, sem):
        for c in copies(e, gb, ub, db, sem):
            c.start()

    def wait(e, gb, ub, db, sem):
        for c in copies(e, gb, ub, db, sem):
            c.wait()

    @pl.when(m == 0)
    def _():
        issue(eid, g0, u0, d0, sem0)

    @pl.when(jnp.logical_and(newf == 1, dpar == 0))
    def _():
        wait(eid, g0, u0, d0, sem0)

        @pl.when(nxt < NE)
        def _():
            issue(nxt, g1, u1, d1, sem1)

    @pl.when(jnp.logical_and(newf == 1, dpar == 1))
    def _():
        wait(eid, g1, u1, d1, sem1)

        @pl.when(nxt < NE)
        def _():
            issue(nxt, g0, u0, d0, sem0)

    def tile(gb, ub, db):
        x = x_ref[...]
        g = lax.dot_general(x, gb[...], (((1,), (1,)), ((), ())),
                            preferred_element_type=jnp.float32)
        u = lax.dot_general(x, ub[...], (((1,), (1,)), ((), ())),
                            preferred_element_type=jnp.float32)
        a = g * (1.0 / (1.0 + jnp.exp(-g))) * u
        y = lax.dot_general(a, db[...], (((1,), (1,)), ((), ())),
                            preferred_element_type=jnp.float32)
        out_ref[...] = y * ws_ref[:, 0:1]

    @pl.when(jnp.logical_and(skip == 0, dpar == 0))
    def _():
        tile(g0, u0, d0)

    @pl.when(jnp.logical_and(skip == 0, dpar == 1))
    def _():
        tile(g1, u1, d1)


def _ffn_call(meta, xs, ws, gate_w, up_w, down_w):
    grid_spec = pltpu.PrefetchScalarGridSpec(
        num_scalar_prefetch=1,
        grid=(NT,),
        in_specs=[
            pl.BlockSpec((SUB, HIDDEN), lambda m, meta: (m, 0)),
            pl.BlockSpec((SUB, 128), lambda m, meta: (m, 0)),
            pl.BlockSpec(memory_space=pl.ANY),
            pl.BlockSpec(memory_space=pl.ANY),
            pl.BlockSpec(memory_space=pl.ANY),
        ],
        out_specs=pl.BlockSpec((SUB, HIDDEN), lambda m, meta: (m, 0)),
        scratch_shapes=[
            pltpu.VMEM((INTER, HIDDEN), jnp.float32),
            pltpu.VMEM((INTER, HIDDEN), jnp.float32),
            pltpu.VMEM((HIDDEN, INTER), jnp.float32),
            pltpu.VMEM((INTER, HIDDEN), jnp.float32),
            pltpu.VMEM((INTER, HIDDEN), jnp.float32),
            pltpu.VMEM((HIDDEN, INTER), jnp.float32),
            pltpu.SemaphoreType.DMA,
            pltpu.SemaphoreType.DMA,
        ],
    )
    return pl.pallas_call(
        _ffn_body,
        grid_spec=grid_spec,
        out_shape=jax.ShapeDtypeStruct((TP, HIDDEN), jnp.float32),
        compiler_params=pltpu.CompilerParams(
            dimension_semantics=("arbitrary",),
            vmem_limit_bytes=63 * 1024 * 1024,
        ),
    )(meta, xs, ws, gate_w, up_w, down_w)


def _combine_body(ys_hbm, pos_hbm, out_hbm, idx_v, rows_v):
    wid = lax.axis_index("s") * NC + lax.axis_index("c")
    base0 = wid * TPW
    for c in range(NCHUNK):
        base = base0 + c * CHUNK
        pltpu.sync_copy(pos_hbm.at[pl.ds(base, CHUNK)], idx_v)
        pltpu.sync_copy(ys_hbm.at[idx_v], rows_v)
        pltpu.sync_copy(rows_v, out_hbm.at[pl.ds(base, CHUNK)])


def _combine_call(ys, pos1):
    mesh = plsc.VectorSubcoreMesh(core_axis_name="c", subcore_axis_name="s")
    f = pl.kernel(
        _combine_body,
        out_type=jax.ShapeDtypeStruct((T, HIDDEN), jnp.float32),
        mesh=mesh,
        scratch_types=[
            pltpu.VMEM((CHUNK,), jnp.int32),
            pltpu.VMEM((CHUNK, HIDDEN), jnp.float32),
        ],
    )
    return f(ys, pos1)


def kernel(hidden_states, router_w, gate_w, up_w, down_w):
    bsz, seq, h = hidden_states.shape
    xf = hidden_states.reshape(T, h)
    pos1, w16, meta = _router_call(xf, router_w)
    xs, ws = _dispatch_call(xf, w16, pos1)
    ys = _ffn_call(meta, xs, ws, gate_w, up_w, down_w)
    out = _combine_call(ys, pos1)
    return out.reshape(bsz, seq, h)
